# Initial kernel scaffold; baseline (speedup 1.0000x reference)
import jax
import jax.numpy as jnp
from jax.experimental import pallas as pl


def kernel(x, encoder, pos_emb):
    raise NotImplementedError("write your pallas kernel here")


# speedup vs baseline: 1.2206x; 1.2206x over previous
import functools

import jax
import jax.numpy as jnp
from jax import lax
from jax.experimental import pallas as pl
from jax.experimental.pallas import tpu as pltpu
from jax.experimental.pallas import tpu_sc as plsc

SEQ = 200
BATCH = 1024
EMB = 64
C = 128
NLANES = 16
NW = 32
TILES_PER_POS = BATCH // C
TOTAL_TILES = SEQ * TILES_PER_POS
PER_W = TOTAL_TILES // NW
ROW_UNROLL = 4


def _emb_kernel(x_hbm, enc_hbm, pos_hbm, out_hbm, idx_v, rows_v, pos_v, sem):
    cid = lax.axis_index("c")
    sid = lax.axis_index("s")
    wid = sid * 2 + cid

    def step(t, carry):
        tau = wid * PER_W + t
        s = tau // TILES_PER_POS
        b0 = (tau % TILES_PER_POS) * C
        pltpu.sync_copy(x_hbm.at[s, pl.ds(b0, C)], idx_v)
        pltpu.sync_copy(pos_hbm.at[s], pos_v)
        pltpu.async_copy(enc_hbm.at[idx_v], rows_v, sem).wait()
        p = [pos_v[pl.ds(j * NLANES, NLANES)] for j in range(EMB // NLANES)]

        def addrows(i, c2):
            for u in range(ROW_UNROLL):
                r = i * ROW_UNROLL + u
                for j in range(EMB // NLANES):
                    sl = pl.ds(j * NLANES, NLANES)
                    rows_v[r, sl] = rows_v[r, sl] + p[j]
            return c2

        lax.fori_loop(0, C // ROW_UNROLL, addrows, 0)
        pltpu.sync_copy(rows_v, out_hbm.at[s, pl.ds(b0, C)])
        return carry

    lax.fori_loop(0, PER_W, step, 0)


def kernel(x, encoder, pos_emb):
    mesh = plsc.VectorSubcoreMesh(core_axis_name="c", subcore_axis_name="s")
    run = functools.partial(
        pl.kernel,
        mesh=mesh,
        out_type=jax.ShapeDtypeStruct((SEQ, BATCH, EMB), jnp.float32),
        scratch_types=[
            pltpu.VMEM((C,), jnp.int32),
            pltpu.VMEM((C, EMB), jnp.float32),
            pltpu.VMEM((EMB,), jnp.float32),
            pltpu.SemaphoreType.DMA,
        ],
        compiler_params=pltpu.CompilerParams(use_tc_tiling_on_sc=False),
    )(_emb_kernel)
    return run(x, encoder, pos_emb)


# speedup vs baseline: 1.3412x; 1.0988x over previous
import functools

import jax
import jax.numpy as jnp
from jax import lax
from jax.experimental import pallas as pl
from jax.experimental.pallas import tpu as pltpu
from jax.experimental.pallas import tpu_sc as plsc

SEQ = 200
BATCH = 1024
EMB = 64
C = 128
NLANES = 16
NW = 32
TILES_PER_POS = BATCH // C
TOTAL_TILES = SEQ * TILES_PER_POS
PER_W = TOTAL_TILES // NW
ROW_UNROLL = 4
NPOS = 8


def _emb_kernel(x_hbm, enc_hbm, pos_hbm, out_hbm,
                idx_all, rows, pos_all, sem0, sem1):
    cid = lax.axis_index("c")
    sid = lax.axis_index("s")
    wid = sid * 2 + cid
    tau0 = wid * PER_W
    s0 = tau0 // TILES_PER_POS

    pltpu.sync_copy(x_hbm.at[pl.ds(tau0, PER_W)], idx_all)
    pltpu.sync_copy(pos_hbm.at[pl.ds(s0, NPOS)], pos_all)

    sems = (sem0, sem1)

    def gather(t, b):
        return pltpu.async_copy(enc_hbm.at[idx_all.at[t]], rows.at[b], sems[b])

    gather(0, 0)

    def pair(i, carry):
        for b in range(2):
            t = 2 * i + b

            @pl.when(t + 1 < PER_W)
            def _():
                gather(t + 1, 1 - b)

            pltpu.make_async_copy(
                enc_hbm.at[idx_all.at[t]], rows.at[b], sems[b]).wait()

            tau = tau0 + t
            s = tau // TILES_PER_POS
            sr = s - s0
            b0 = (tau % TILES_PER_POS) * C
            p = [pos_all[sr, pl.ds(j * NLANES, NLANES)]
                 for j in range(EMB // NLANES)]

            def addrows(k, c2):
                for u in range(ROW_UNROLL):
                    r = k * ROW_UNROLL + u
                    for j in range(EMB // NLANES):
                        sl = pl.ds(j * NLANES, NLANES)
                        rows[b, r, sl] = rows[b, r, sl] + p[j]
                return c2

            lax.fori_loop(0, C // ROW_UNROLL, addrows, 0)
            pltpu.sync_copy(rows.at[b], out_hbm.at[s, pl.ds(b0, C)])
        return carry

    lax.fori_loop(0, PER_W // 2, pair, 0)


def kernel(x, encoder, pos_emb):
    mesh = plsc.VectorSubcoreMesh(core_axis_name="c", subcore_axis_name="s")
    run = functools.partial(
        pl.kernel,
        mesh=mesh,
        out_type=jax.ShapeDtypeStruct((SEQ, BATCH, EMB), jnp.float32),
        scratch_types=[
            pltpu.VMEM((PER_W, C), jnp.int32),
            pltpu.VMEM((2, C, EMB), jnp.float32),
            pltpu.VMEM((NPOS, EMB), jnp.float32),
            pltpu.SemaphoreType.DMA,
            pltpu.SemaphoreType.DMA,
        ],
        compiler_params=pltpu.CompilerParams(use_tc_tiling_on_sc=False),
    )(_emb_kernel)
    return run(x.reshape(TOTAL_TILES, C), encoder, pos_emb)
